# baseline (speedup 1.0000x reference)
import jax
import jax.numpy as jnp
from jax.experimental import pallas as pl
from jax.experimental.pallas import tpu as pltpu

_NEGPOS_RATIO = 3
_NP_CHUNKS = 8


def _body(cta_ref, ctb_ref, conf_ref, loc_ref, loct_ref,
          out_l_ref, out_c_ref, out_n_ref,
          lc_s, np_s, plc_s, *, num_priors):
    b = pl.program_id(0)
    ip = pl.program_id(1)
    nchunks = pl.num_programs(1)

    @pl.when(jnp.logical_and(b == 0, ip == 0))
    def _():
        out_l_ref[...] = jnp.zeros((1, 1), jnp.float32)
        out_c_ref[...] = jnp.zeros((1, 1), jnp.float32)
        out_n_ref[...] = jnp.zeros((1, 1), jnp.float32)

    @pl.when(ip == 0)
    def _():
        np_s[0] = 0
        plc_s[0] = 0.0

    conf = conf_ref[0]
    ct_a = cta_ref[0]
    ct_b = ctb_ref[0, 0]
    PB, C = conf.shape
    ones = jnp.ones((1, C), jnp.float32)
    contract = (((1,), (1,)), ((), ()))

    m = jnp.max(conf)
    e = jnp.exp(conf - m)
    s = jax.lax.dot_general(ones, e, contract,
                            preferred_element_type=jnp.float32)
    cls = jax.lax.broadcasted_iota(jnp.int32, (PB, C), 1)
    g = jax.lax.dot_general(ones, jnp.where(cls == ct_a, conf - m, 0.0),
                            contract,
                            preferred_element_type=jnp.float32)
    lc = jnp.log(s) - g

    pos_l = ct_b > 0
    np_chunk = jnp.sum(pos_l.astype(jnp.int32))
    np_s[0] += np_chunk
    plc_s[0] += jnp.sum(jnp.where(pos_l, lc, 0.0))
    lcm = jnp.where(pos_l, 0.0, jnp.maximum(lc, 0.0))
    lc_s[pl.ds(ip, 1), :] = lcm

    pos_s = (ct_a > 0).astype(jnp.float32)
    d = loc_ref[0] - loct_ref[0]
    ad = jnp.abs(d)
    sl1 = jnp.where(ad < 1.0, 0.5 * d * d, ad - 0.5)
    out_l_ref[...] += jnp.sum(sl1 * pos_s, keepdims=True)

    @pl.when(ip == nchunks - 1)
    def _():
        np_row = np_s[0]
        k = jnp.minimum(_NEGPOS_RATIO * np_row, num_priors - 1)
        lcs = lc_s[...]
        bits = jax.lax.bitcast_convert_type(lcs, jnp.int32)

        def step(i, t):
            cand = t | (jnp.int32(1) << (jnp.int32(30) - i))
            cnt = jnp.sum((bits >= cand).astype(jnp.int32))
            return jnp.where(cnt >= k, cand, t)

        t = jax.lax.fori_loop(0, 31, step, jnp.int32(0))
        neg_lc = jnp.sum(jnp.where(bits >= t, lcs, 0.0), keepdims=True)
        out_c_ref[...] += neg_lc + jnp.full((1, 1), plc_s[0])
        out_n_ref[...] += jnp.full((1, 1), np_row.astype(jnp.float32))


@jax.jit
def kernel(arm_loc_data, arm_conf_data, loc_t, conf_t):
    B, P, C = arm_conf_data.shape
    nchunks = _NP_CHUNKS
    PB = P // nchunks
    ct_a = conf_t[:, :, None]
    ct_b = conf_t.reshape(B, nchunks, 1, PB)
    import functools
    body = functools.partial(_body, num_priors=P)
    outs = pl.pallas_call(
        body,
        grid=(B, nchunks),
        in_specs=[
            pl.BlockSpec((1, PB, 1), lambda b, i: (b, i, 0)),
            pl.BlockSpec((1, 1, 1, PB), lambda b, i: (b, i, 0, 0)),
            pl.BlockSpec((1, PB, C), lambda b, i: (b, i, 0)),
            pl.BlockSpec((1, PB, 4), lambda b, i: (b, i, 0)),
            pl.BlockSpec((1, PB, 4), lambda b, i: (b, i, 0)),
        ],
        out_specs=[
            pl.BlockSpec((1, 1), lambda b, i: (0, 0)),
            pl.BlockSpec((1, 1), lambda b, i: (0, 0)),
            pl.BlockSpec((1, 1), lambda b, i: (0, 0)),
        ],
        out_shape=[jax.ShapeDtypeStruct((1, 1), jnp.float32)] * 3,
        scratch_shapes=[
            pltpu.VMEM((nchunks, PB), jnp.float32),
            pltpu.SMEM((1,), jnp.int32),
            pltpu.SMEM((1,), jnp.float32),
        ],
    )(ct_a, ct_b, arm_conf_data, arm_loc_data, loc_t)
    l, c, n = outs
    nn = n[0, 0]
    return (l[0, 0] / nn, c[0, 0] / nn)


# speedup vs baseline: 2.4265x; 2.4265x over previous
import functools

import jax
import jax.numpy as jnp
from jax.experimental import pallas as pl
from jax.experimental.pallas import tpu as pltpu

_NEGPOS_RATIO = 3
_NP_CHUNKS = 8


def _body(ct_ref, conf_ref, loc_ref, loct_ref,
          out_l_ref, out_c_ref, out_n_ref,
          lc_s, np_s, plc_s, *, num_priors):
    b = pl.program_id(0)
    ip = pl.program_id(1)
    nchunks = pl.num_programs(1)

    @pl.when(jnp.logical_and(b == 0, ip == 0))
    def _():
        out_l_ref[...] = jnp.zeros((1, 1), jnp.float32)
        out_c_ref[...] = jnp.zeros((1, 1), jnp.float32)
        out_n_ref[...] = jnp.zeros((1, 1), jnp.float32)

    @pl.when(ip == 0)
    def _():
        np_s[0] = 0
        plc_s[0] = 0.0

    conf = conf_ref[0, 0]
    ct = ct_ref[0, 0]
    C, PB = conf.shape

    pos = ct > 0
    np_s[0] += jnp.sum(pos.astype(jnp.int32))

    m = jnp.max(conf, axis=0, keepdims=True)
    s = jnp.sum(jnp.exp(conf - m), axis=0, keepdims=True)
    lse = jnp.log(s) + m
    cls = jax.lax.broadcasted_iota(jnp.int32, (C, PB), 0)
    g = jnp.sum(jnp.where(cls == ct, conf, 0.0), axis=0, keepdims=True)
    lc = lse - g

    plc_s[0] += jnp.sum(jnp.where(pos, lc, 0.0))
    lcm = jnp.where(pos, 0.0, jnp.maximum(lc, 0.0))
    lc_s[pl.ds(ip, 1), :] = lcm

    d = loc_ref[0, 0] - loct_ref[0, 0]
    ad = jnp.abs(d)
    sl1 = jnp.where(ad < 1.0, 0.5 * d * d, ad - 0.5)
    posf = pos.astype(jnp.float32)
    out_l_ref[...] += jnp.sum(sl1 * posf, keepdims=True)

    @pl.when(ip == nchunks - 1)
    def _():
        np_row = np_s[0]
        k = jnp.minimum(_NEGPOS_RATIO * np_row, num_priors - 1)
        lcs = lc_s[...]
        bits = jax.lax.bitcast_convert_type(lcs, jnp.int32)

        def step(i, t):
            cand = t | (jnp.int32(1) << (jnp.int32(30) - i))
            cnt = jnp.sum((bits >= cand).astype(jnp.int32))
            return jnp.where(cnt >= k, cand, t)

        t = jax.lax.fori_loop(0, 31, step, jnp.int32(0))
        neg_lc = jnp.sum(jnp.where(bits >= t, lcs, 0.0), keepdims=True)
        out_c_ref[...] += neg_lc + jnp.full((1, 1), plc_s[0])
        out_n_ref[...] += jnp.full((1, 1), np_row.astype(jnp.float32))


@jax.jit
def kernel(arm_loc_data, arm_conf_data, loc_t, conf_t):
    B, P, C = arm_conf_data.shape
    nchunks = _NP_CHUNKS
    PB = P // nchunks
    ct_r = conf_t.reshape(B, nchunks, 1, PB)
    conf_r = arm_conf_data.reshape(B, nchunks, PB, C).transpose(0, 1, 3, 2)
    loc_r = arm_loc_data.reshape(B, nchunks, PB, 4).transpose(0, 1, 3, 2)
    loct_r = loc_t.reshape(B, nchunks, PB, 4).transpose(0, 1, 3, 2)
    body = functools.partial(_body, num_priors=P)
    outs = pl.pallas_call(
        body,
        grid=(B, nchunks),
        in_specs=[
            pl.BlockSpec((1, 1, 1, PB), lambda b, i: (b, i, 0, 0)),
            pl.BlockSpec((1, 1, C, PB), lambda b, i: (b, i, 0, 0)),
            pl.BlockSpec((1, 1, 4, PB), lambda b, i: (b, i, 0, 0)),
            pl.BlockSpec((1, 1, 4, PB), lambda b, i: (b, i, 0, 0)),
        ],
        out_specs=[
            pl.BlockSpec((1, 1), lambda b, i: (0, 0)),
            pl.BlockSpec((1, 1), lambda b, i: (0, 0)),
            pl.BlockSpec((1, 1), lambda b, i: (0, 0)),
        ],
        out_shape=[jax.ShapeDtypeStruct((1, 1), jnp.float32)] * 3,
        scratch_shapes=[
            pltpu.VMEM((nchunks, PB), jnp.float32),
            pltpu.SMEM((1,), jnp.int32),
            pltpu.SMEM((1,), jnp.float32),
        ],
    )(ct_r, conf_r, loc_r, loct_r)
    l, c, n = outs
    nn = n[0, 0]
    return (l[0, 0] / nn, c[0, 0] / nn)


# speedup vs baseline: 2.8013x; 1.1545x over previous
import functools

import jax
import jax.numpy as jnp
from jax.experimental import pallas as pl
from jax.experimental.pallas import tpu as pltpu

_NEGPOS_RATIO = 3
_NP_CHUNKS = 2


def _body(ct_ref, conf_ref, loc_ref, loct_ref,
          out_l_ref, out_c_ref, out_n_ref,
          lc_s, np_s, plc_s, *, num_priors):
    b = pl.program_id(0)
    ip = pl.program_id(1)
    nchunks = pl.num_programs(1)

    @pl.when(jnp.logical_and(b == 0, ip == 0))
    def _():
        out_l_ref[...] = jnp.zeros((1, 1), jnp.float32)
        out_c_ref[...] = jnp.zeros((1, 1), jnp.float32)
        out_n_ref[...] = jnp.zeros((1, 1), jnp.float32)

    @pl.when(ip == 0)
    def _():
        np_s[0] = 0
        plc_s[0] = 0.0

    conf = conf_ref[0, 0]
    ct = ct_ref[0, 0]
    C, PB = conf.shape

    pos = ct > 0
    np_s[0] += jnp.sum(pos.astype(jnp.int32))

    m = jnp.max(conf, axis=0, keepdims=True)
    s = jnp.sum(jnp.exp(conf - m), axis=0, keepdims=True)
    lse = jnp.log(s) + m
    cls = jax.lax.broadcasted_iota(jnp.int32, (C, PB), 0)
    g = jnp.sum(jnp.where(cls == ct, conf, 0.0), axis=0, keepdims=True)
    lc = lse - g

    plc_s[0] += jnp.sum(jnp.where(pos, lc, 0.0))
    lcm = jnp.where(pos, 0.0, jnp.maximum(lc, 0.0))
    lc_s[pl.ds(ip, 1), :] = lcm

    d = loc_ref[0, 0] - loct_ref[0, 0]
    ad = jnp.abs(d)
    sl1 = jnp.where(ad < 1.0, 0.5 * d * d, ad - 0.5)
    posf = pos.astype(jnp.float32)
    out_l_ref[...] += jnp.sum(sl1 * posf, keepdims=True)

    @pl.when(ip == nchunks - 1)
    def _():
        np_row = np_s[0]
        k = jnp.minimum(_NEGPOS_RATIO * np_row, num_priors - 1)
        lcs = lc_s[...]
        bits = jax.lax.bitcast_convert_type(lcs, jnp.int32)

        def step(i, t):
            cand = t | (jnp.int32(1) << (jnp.int32(30) - i))
            cnt = jnp.sum((bits >= cand).astype(jnp.int32))
            return jnp.where(cnt >= k, cand, t)

        t = jax.lax.fori_loop(0, 31, step, jnp.int32(0))
        neg_lc = jnp.sum(jnp.where(bits >= t, lcs, 0.0), keepdims=True)
        out_c_ref[...] += neg_lc + jnp.full((1, 1), plc_s[0])
        out_n_ref[...] += jnp.full((1, 1), np_row.astype(jnp.float32))


@jax.jit
def kernel(arm_loc_data, arm_conf_data, loc_t, conf_t):
    B, P, C = arm_conf_data.shape
    nchunks = _NP_CHUNKS
    PB = P // nchunks
    ct_r = conf_t.reshape(B, nchunks, 1, PB)
    conf_r = arm_conf_data.reshape(B, nchunks, PB, C).transpose(0, 1, 3, 2)
    loc_r = arm_loc_data.reshape(B, nchunks, PB, 4).transpose(0, 1, 3, 2)
    loct_r = loc_t.reshape(B, nchunks, PB, 4).transpose(0, 1, 3, 2)
    body = functools.partial(_body, num_priors=P)
    outs = pl.pallas_call(
        body,
        grid=(B, nchunks),
        in_specs=[
            pl.BlockSpec((1, 1, 1, PB), lambda b, i: (b, i, 0, 0)),
            pl.BlockSpec((1, 1, C, PB), lambda b, i: (b, i, 0, 0)),
            pl.BlockSpec((1, 1, 4, PB), lambda b, i: (b, i, 0, 0)),
            pl.BlockSpec((1, 1, 4, PB), lambda b, i: (b, i, 0, 0)),
        ],
        out_specs=[
            pl.BlockSpec((1, 1), lambda b, i: (0, 0)),
            pl.BlockSpec((1, 1), lambda b, i: (0, 0)),
            pl.BlockSpec((1, 1), lambda b, i: (0, 0)),
        ],
        out_shape=[jax.ShapeDtypeStruct((1, 1), jnp.float32)] * 3,
        scratch_shapes=[
            pltpu.VMEM((nchunks, PB), jnp.float32),
            pltpu.SMEM((1,), jnp.int32),
            pltpu.SMEM((1,), jnp.float32),
        ],
    )(ct_r, conf_r, loc_r, loct_r)
    l, c, n = outs
    nn = n[0, 0]
    return (l[0, 0] / nn, c[0, 0] / nn)
